# baseline (speedup 1.0000x reference)
import functools
import jax
import jax.numpy as jnp
import numpy as np
from jax import lax
from jax.experimental import pallas as pl
from jax.experimental.pallas import tpu as pltpu
from jax.experimental.pallas import tpu_sc as plsc

H = 256

NHI = 81
BAND = 256
NW = 32
CHUNK = 64
DEPTH = 4
AHEAD = 2


def _tbl_body(w0, w1, w2, w3, w4, w5, w6, w7, w8, out):
    i = pl.program_id(0)
    acc = w0[...]
    for wref in (w1, w2, w3, w4):
        w = wref[...]
        acc = jnp.concatenate(
            [acc + w[0:1, :], acc + w[1:2, :], acc + w[2:3, :]], axis=0)
    hi = jnp.zeros((1, H), jnp.float32)
    r = i
    for wref in (w6, w7, w8):
        w = wref[...]
        d = r % 3
        r = r // 3
        hi = hi + jnp.where(d == 0, w[0:1, :],
                            jnp.where(d == 1, w[1:2, :], w[2:3, :]))
    acc = acc + hi
    w5v = w5[...]
    pad = jnp.zeros((BAND - 243, H), jnp.float32)
    bands = []
    for k in range(3):
        bands += [acc + w5v[k:k + 1, :], pad]
    v = jnp.concatenate(bands, axis=0) * 16.0
    a = lax.bitcast_convert_type(
        v[:, :H // 2].astype(jnp.bfloat16), jnp.uint16)
    b = lax.bitcast_convert_type(
        v[:, H // 2:].astype(jnp.bfloat16), jnp.uint16)
    out[...] = (a.astype(jnp.int32) |
                jnp.left_shift(b.astype(jnp.int32), 16))


def _build_table(ws, interpret=False):
    return pl.pallas_call(
        _tbl_body,
        grid=(NHI // 3,),
        in_specs=[pl.BlockSpec((3, H), lambda i: (0, 0))] * 9,
        out_specs=pl.BlockSpec((3 * BAND, H // 2), lambda i: (i, 0)),
        out_shape=jax.ShapeDtypeStruct((NHI * BAND, H // 2), jnp.int32),
        interpret=interpret,
    )(*ws)


def _sc_body(n_tok, i0, i1, i2, i3, i4, i5, i6, i7, i8, table, out,
             b0, b1, b2, b3, b4, b5, b6, b7, b8, codes,
             rows0, rows1, rows2, rows3, f0buf, f1buf,
             isem, gsem0, gsem1, gsem2, gsem3, ssem0, ssem1):
    per_w = n_tok // NW
    nchunk = per_w // CHUNK
    wid = lax.axis_index("s") * 2 + lax.axis_index("c")
    base = wid * per_w
    t_len = i0.shape[1]
    rows_per_w = per_w // t_len
    row0 = wid * rows_per_w
    bufs = (b0, b1, b2, b3, b4, b5, b6, b7, b8)
    stages = []
    for src, dst in zip((i0, i1, i2, i3, i4, i5, i6, i7, i8), bufs):
        for r in range(rows_per_w):
            stages.append(pltpu.async_copy(
                src.at[row0 + r], dst.at[pl.ds(r * t_len, t_len)], isem))
    for h in stages:
        h.wait()

    rows = (rows0, rows1, rows2, rows3)
    fbufs = (f0buf, f1buf)
    gsem = (gsem0, gsem1, gsem2, gsem3)
    ssem = (ssem0, ssem1)

    def code_chunk(g):
        for j in range(CHUNK // 16):
            o = g * CHUNK + j * 16
            d = [b[pl.ds(o, 16)] for b in bufs]
            hi = ((d[8] * 3 + d[7]) * 3 + d[6]) * 3 + d[5]
            lo = (((d[4] * 3 + d[3]) * 3 + d[2]) * 3 + d[1]) * 3 + d[0]
            codes[pl.ds(o, 16)] = hi * BAND + lo

    def gather_start(g, s):
        pltpu.async_copy(table.at[codes.at[pl.ds(g * CHUNK, CHUNK)]],
                         rows[s], gsem[s])

    def gather_wait(g, s):
        pltpu.make_async_copy(table.at[codes.at[pl.ds(g * CHUNK, CHUNK)]],
                              rows[s], gsem[s]).wait()

    def scatter_start(g, f):
        pltpu.async_copy(fbufs[f], out.at[pl.ds(base + g * CHUNK, CHUNK)],
                         ssem[f])

    def scatter_wait(g, f):
        pltpu.make_async_copy(fbufs[f],
                              out.at[pl.ds(base + g * CHUNK, CHUNK)],
                              ssem[f]).wait()

    def convert(s, f):
        ibuf, fbuf = rows[s], fbufs[f]

        @pl.loop(0, CHUNK)
        def row_loop(r):
            for v in range(H // 32):
                x = ibuf[r, pl.ds(16 * v, 16)]
                msk = jnp.full((16,), -65536, jnp.int32)
                lo = lax.bitcast_convert_type(jnp.left_shift(x, 16),
                                              jnp.float32)
                hi = lax.bitcast_convert_type(jnp.bitwise_and(x, msk),
                                              jnp.float32)
                fbuf[r, pl.ds(16 * v, 16)] = lo
                fbuf[r, pl.ds(H // 2 + 16 * v, 16)] = hi

    for g in range(AHEAD):
        code_chunk(g)
        gather_start(g, g % DEPTH)

    @pl.loop(0, nchunk, step=DEPTH)
    def chunk_loop(k):
        for b in range(DEPTH):
            g = k + b
            s = b
            f = b % 2
            gather_wait(g, s)

            @pl.when(g - 2 >= 0)
            def _():
                scatter_wait(g - 2, f)

            convert(s, f)
            scatter_start(g, f)
            nxt = g + AHEAD
            s2 = (b + AHEAD) % DEPTH

            @pl.when(nxt < nchunk)
            def _():
                code_chunk(nxt)
                gather_start(nxt, s2)

    for d in (2, 1):
        scatter_wait(nchunk - d, (nchunk - d) % 2)


def _sc_lookup(idxs, table, interpret=False):
    n_tok = idxs[0].shape[0] * idxs[0].shape[1]
    per_w = n_tok // NW
    mesh = plsc.VectorSubcoreMesh(core_axis_name="c", subcore_axis_name="s")
    scratch = [pltpu.VMEM((per_w,), jnp.int32) for _ in range(9)]
    scratch += [pltpu.VMEM((per_w,), jnp.int32)]
    scratch += [pltpu.VMEM((CHUNK, H // 2), jnp.int32) for _ in range(DEPTH)]
    scratch += [pltpu.VMEM((CHUNK, H), jnp.float32) for _ in range(2)]
    scratch += [pltpu.SemaphoreType.DMA for _ in range(DEPTH + 3)]
    fn = pl.kernel(
        functools.partial(_sc_body, n_tok),
        out_type=jax.ShapeDtypeStruct((n_tok, H), jnp.float32),
        mesh=mesh,
        scratch_types=scratch,
        interpret=interpret,
    )
    return fn(*idxs, table)


def kernel(mix, falsetto, breathy, bubble, strong, weak, pharyngeal,
           vibrato, glissando,
           W_mix, W_falsetto, W_breathy, W_bubble, W_strong, W_weak,
           W_pharyngeal, W_vibrato, W_glissando):
    b, t = mix.shape
    idxs = [mix, falsetto, breathy, bubble, strong, weak, pharyngeal,
            vibrato, glissando]
    ws = (W_mix, W_falsetto, W_breathy, W_bubble, W_strong, W_weak,
          W_pharyngeal, W_vibrato, W_glissando)
    table = _build_table(ws)
    out = _sc_lookup(idxs, table)
    return out.reshape(b, t, H)


# speedup vs baseline: 1.0042x; 1.0042x over previous
import functools
import jax
import jax.numpy as jnp
import numpy as np
from jax import lax
from jax.experimental import pallas as pl
from jax.experimental.pallas import tpu as pltpu
from jax.experimental.pallas import tpu_sc as plsc

H = 256

NHI = 81
BAND = 256
NW = 32
CHUNK = 64
DEPTH = 4
AHEAD = 2


def _tbl_body(w0, w1, w2, w3, w4, w5, w6, w7, w8, out):
    i = pl.program_id(0)
    acc = w0[...]
    for wref in (w1, w2, w3, w4):
        w = wref[...]
        acc = jnp.concatenate(
            [acc + w[0:1, :], acc + w[1:2, :], acc + w[2:3, :]], axis=0)
    hi = jnp.zeros((1, H), jnp.float32)
    r = i
    for wref in (w6, w7, w8):
        w = wref[...]
        d = r % 3
        r = r // 3
        hi = hi + jnp.where(d == 0, w[0:1, :],
                            jnp.where(d == 1, w[1:2, :], w[2:3, :]))
    acc = acc + hi
    w5v = w5[...]
    pad = jnp.zeros((BAND - 243, H), jnp.float32)
    bands = []
    for k in range(3):
        bands += [acc + w5v[k:k + 1, :], pad]
    v = jnp.concatenate(bands, axis=0) * 16.0
    a = lax.bitcast_convert_type(
        v[:, :H // 2].astype(jnp.bfloat16), jnp.uint16)
    b = lax.bitcast_convert_type(
        v[:, H // 2:].astype(jnp.bfloat16), jnp.uint16)
    out[...] = (a.astype(jnp.int32) |
                jnp.left_shift(b.astype(jnp.int32), 16))


def _build_table(ws, interpret=False):
    return pl.pallas_call(
        _tbl_body,
        grid=(NHI // 3,),
        in_specs=[pl.BlockSpec((3, H), lambda i: (0, 0))] * 9,
        out_specs=pl.BlockSpec((3 * BAND, H // 2), lambda i: (i, 0)),
        out_shape=jax.ShapeDtypeStruct((NHI * BAND, H // 2), jnp.int32),
        interpret=interpret,
    )(*ws)


def _sc_body(n_tok, i0, i1, i2, i3, i4, i5, i6, i7, i8, table, out,
             b0, b1, b2, b3, b4, b5, b6, b7, b8, codes,
             rows0, rows1, rows2, rows3, f0buf, f1buf,
             isem, gsem0, gsem1, gsem2, gsem3, ssem0, ssem1):
    per_w = n_tok // NW
    nchunk = per_w // CHUNK
    wid = lax.axis_index("s") * 2 + lax.axis_index("c")
    base = wid * per_w
    t_len = i0.shape[1]
    rows_per_w = per_w // t_len
    row0 = wid * rows_per_w
    bufs = (b0, b1, b2, b3, b4, b5, b6, b7, b8)
    stages = []
    for src, dst in zip((i0, i1, i2, i3, i4, i5, i6, i7, i8), bufs):
        for r in range(rows_per_w):
            stages.append(pltpu.async_copy(
                src.at[row0 + r], dst.at[pl.ds(r * t_len, t_len)], isem))
    for h in stages:
        h.wait()

    rows = (rows0, rows1, rows2, rows3)
    fbufs = (f0buf, f1buf)
    gsem = (gsem0, gsem1, gsem2, gsem3)
    ssem = (ssem0, ssem1)

    def code_chunk(g):
        for j in range(CHUNK // 16):
            o = g * CHUNK + j * 16
            d = [b[pl.ds(o, 16)] for b in bufs]
            hi = ((d[8] * 3 + d[7]) * 3 + d[6]) * 3 + d[5]
            lo = (((d[4] * 3 + d[3]) * 3 + d[2]) * 3 + d[1]) * 3 + d[0]
            codes[pl.ds(o, 16)] = hi * BAND + lo

    def gather_start(g, s):
        pltpu.async_copy(table.at[codes.at[pl.ds(g * CHUNK, CHUNK)]],
                         rows[s], gsem[s])

    def gather_wait(g, s):
        pltpu.make_async_copy(table.at[codes.at[pl.ds(g * CHUNK, CHUNK)]],
                              rows[s], gsem[s]).wait()

    def scatter_start(g, f):
        pltpu.async_copy(fbufs[f], out.at[pl.ds(base + g * CHUNK, CHUNK)],
                         ssem[f])

    def scatter_wait(g, f):
        pltpu.make_async_copy(fbufs[f],
                              out.at[pl.ds(base + g * CHUNK, CHUNK)],
                              ssem[f]).wait()

    def convert(s, f):
        ibuf, fbuf = rows[s], fbufs[f]

        msk = jnp.full((16,), -65536, jnp.int32)

        @pl.loop(0, CHUNK, step=8)
        def row_loop(r0):
            for dr in range(8):
                r = r0 + dr
                for v in range(H // 32):
                    x = ibuf[r, pl.ds(16 * v, 16)]
                    lo = lax.bitcast_convert_type(jnp.left_shift(x, 16),
                                                  jnp.float32)
                    hi = lax.bitcast_convert_type(jnp.bitwise_and(x, msk),
                                                  jnp.float32)
                    fbuf[r, pl.ds(16 * v, 16)] = lo
                    fbuf[r, pl.ds(H // 2 + 16 * v, 16)] = hi

    for g in range(AHEAD):
        code_chunk(g)
        gather_start(g, g % DEPTH)

    @pl.loop(0, nchunk, step=DEPTH)
    def chunk_loop(k):
        for b in range(DEPTH):
            g = k + b
            s = b
            f = b % 2
            gather_wait(g, s)

            @pl.when(g - 2 >= 0)
            def _():
                scatter_wait(g - 2, f)

            convert(s, f)
            scatter_start(g, f)
            nxt = g + AHEAD
            s2 = (b + AHEAD) % DEPTH

            @pl.when(nxt < nchunk)
            def _():
                code_chunk(nxt)
                gather_start(nxt, s2)

    for d in (2, 1):
        scatter_wait(nchunk - d, (nchunk - d) % 2)


def _sc_lookup(idxs, table, interpret=False):
    n_tok = idxs[0].shape[0] * idxs[0].shape[1]
    per_w = n_tok // NW
    mesh = plsc.VectorSubcoreMesh(core_axis_name="c", subcore_axis_name="s")
    scratch = [pltpu.VMEM((per_w,), jnp.int32) for _ in range(9)]
    scratch += [pltpu.VMEM((per_w,), jnp.int32)]
    scratch += [pltpu.VMEM((CHUNK, H // 2), jnp.int32) for _ in range(DEPTH)]
    scratch += [pltpu.VMEM((CHUNK, H), jnp.float32) for _ in range(2)]
    scratch += [pltpu.SemaphoreType.DMA for _ in range(DEPTH + 3)]
    fn = pl.kernel(
        functools.partial(_sc_body, n_tok),
        out_type=jax.ShapeDtypeStruct((n_tok, H), jnp.float32),
        mesh=mesh,
        scratch_types=scratch,
        interpret=interpret,
    )
    return fn(*idxs, table)


def kernel(mix, falsetto, breathy, bubble, strong, weak, pharyngeal,
           vibrato, glissando,
           W_mix, W_falsetto, W_breathy, W_bubble, W_strong, W_weak,
           W_pharyngeal, W_vibrato, W_glissando):
    b, t = mix.shape
    idxs = [mix, falsetto, breathy, bubble, strong, weak, pharyngeal,
            vibrato, glissando]
    ws = (W_mix, W_falsetto, W_breathy, W_bubble, W_strong, W_weak,
          W_pharyngeal, W_vibrato, W_glissando)
    table = _build_table(ws)
    out = _sc_lookup(idxs, table)
    return out.reshape(b, t, H)


# speedup vs baseline: 1.4943x; 1.4881x over previous
import functools
import jax
import jax.numpy as jnp
import numpy as np
from jax import lax
from jax.experimental import pallas as pl
from jax.experimental.pallas import tpu as pltpu
from jax.experimental.pallas import tpu_sc as plsc

H = 256

NHI = 81
BAND = 256
NW = 32
CHUNK = 64
DEPTH = 4
AHEAD = 2


def _tbl_body(w0, w1, w2, w3, w4, w5, w6, w7, w8, out):
    i = pl.program_id(0)
    acc = w0[...]
    for wref in (w1, w2, w3, w4):
        w = wref[...]
        acc = jnp.concatenate(
            [acc + w[0:1, :], acc + w[1:2, :], acc + w[2:3, :]], axis=0)
    hi = jnp.zeros((1, H), jnp.float32)
    r = i
    for wref in (w6, w7, w8):
        w = wref[...]
        d = r % 3
        r = r // 3
        hi = hi + jnp.where(d == 0, w[0:1, :],
                            jnp.where(d == 1, w[1:2, :], w[2:3, :]))
    acc = acc + hi
    w5v = w5[...]
    pad = jnp.zeros((BAND - 243, H), jnp.float32)
    bands = []
    for k in range(3):
        bands += [acc + w5v[k:k + 1, :], pad]
    out[...] = jnp.concatenate(bands, axis=0) * 16.0


def _build_table(ws, interpret=False):
    return pl.pallas_call(
        _tbl_body,
        grid=(NHI // 3,),
        in_specs=[pl.BlockSpec((3, H), lambda i: (0, 0))] * 9,
        out_specs=pl.BlockSpec((3 * BAND, H), lambda i: (i, 0)),
        out_shape=jax.ShapeDtypeStruct((NHI * BAND, H), jnp.float32),
        interpret=interpret,
    )(*ws)


def _sc_body(n_tok, i0, i1, i2, i3, i4, i5, i6, i7, i8, table, out,
             b0, b1, b2, b3, b4, b5, b6, b7, b8, codes,
             rows0, rows1, rows2, rows3,
             isem, gsem0, gsem1, gsem2, gsem3, ssem0, ssem1, ssem2, ssem3):
    per_w = n_tok // NW
    nchunk = per_w // CHUNK
    wid = lax.axis_index("s") * 2 + lax.axis_index("c")
    base = wid * per_w
    t_len = i0.shape[1]
    rows_per_w = per_w // t_len
    row0 = wid * rows_per_w
    bufs = (b0, b1, b2, b3, b4, b5, b6, b7, b8)
    stages = []
    for src, dst in zip((i0, i1, i2, i3, i4, i5, i6, i7, i8), bufs):
        for r in range(rows_per_w):
            stages.append(pltpu.async_copy(
                src.at[row0 + r], dst.at[pl.ds(r * t_len, t_len)], isem))
    for h in stages:
        h.wait()

    rows = (rows0, rows1, rows2, rows3)
    gsem = (gsem0, gsem1, gsem2, gsem3)
    ssem = (ssem0, ssem1, ssem2, ssem3)

    def code_chunk(g):
        for j in range(CHUNK // 16):
            o = g * CHUNK + j * 16
            d = [b[pl.ds(o, 16)] for b in bufs]
            hi = ((d[8] * 3 + d[7]) * 3 + d[6]) * 3 + d[5]
            lo = (((d[4] * 3 + d[3]) * 3 + d[2]) * 3 + d[1]) * 3 + d[0]
            codes[pl.ds(o, 16)] = hi * BAND + lo

    def gather_start(g, s):
        pltpu.async_copy(table.at[codes.at[pl.ds(g * CHUNK, CHUNK)]],
                         rows[s], gsem[s])

    def gather_wait(g, s):
        pltpu.make_async_copy(table.at[codes.at[pl.ds(g * CHUNK, CHUNK)]],
                              rows[s], gsem[s]).wait()

    def scatter_start(g, s):
        pltpu.async_copy(rows[s], out.at[pl.ds(base + g * CHUNK, CHUNK)],
                         ssem[s])

    def scatter_wait(g, s):
        pltpu.make_async_copy(rows[s], out.at[pl.ds(base + g * CHUNK, CHUNK)],
                              ssem[s]).wait()

    for g in range(AHEAD):
        code_chunk(g)
        gather_start(g, g % DEPTH)

    @pl.loop(0, nchunk, step=DEPTH)
    def chunk_loop(k):
        for b in range(DEPTH):
            g = k + b
            s = b
            gather_wait(g, s)
            scatter_start(g, s)
            nxt = g + AHEAD
            s2 = (b + AHEAD) % DEPTH

            @pl.when(nxt < nchunk)
            def _():
                code_chunk(nxt)

                @pl.when(nxt - DEPTH >= 0)
                def _():
                    scatter_wait(nxt - DEPTH, s2)

                gather_start(nxt, s2)

    for s in range(DEPTH):
        scatter_wait(nchunk - DEPTH + s, s)


def _sc_lookup(idxs, table, interpret=False):
    n_tok = idxs[0].shape[0] * idxs[0].shape[1]
    per_w = n_tok // NW
    mesh = plsc.VectorSubcoreMesh(core_axis_name="c", subcore_axis_name="s")
    scratch = [pltpu.VMEM((per_w,), jnp.int32) for _ in range(9)]
    scratch += [pltpu.VMEM((per_w,), jnp.int32)]
    scratch += [pltpu.VMEM((CHUNK, H), jnp.float32) for _ in range(DEPTH)]
    scratch += [pltpu.SemaphoreType.DMA for _ in range(2 * DEPTH + 1)]
    fn = pl.kernel(
        functools.partial(_sc_body, n_tok),
        out_type=jax.ShapeDtypeStruct((n_tok, H), jnp.float32),
        mesh=mesh,
        scratch_types=scratch,
        interpret=interpret,
    )
    return fn(*idxs, table)


def kernel(mix, falsetto, breathy, bubble, strong, weak, pharyngeal,
           vibrato, glissando,
           W_mix, W_falsetto, W_breathy, W_bubble, W_strong, W_weak,
           W_pharyngeal, W_vibrato, W_glissando):
    b, t = mix.shape
    idxs = [mix, falsetto, breathy, bubble, strong, weak, pharyngeal,
            vibrato, glissando]
    ws = (W_mix, W_falsetto, W_breathy, W_bubble, W_strong, W_weak,
          W_pharyngeal, W_vibrato, W_glissando)
    table = _build_table(ws)
    out = _sc_lookup(idxs, table)
    return out.reshape(b, t, H)


# speedup vs baseline: 1.5706x; 1.0510x over previous
import functools
import jax
import jax.numpy as jnp
import numpy as np
from jax import lax
from jax.experimental import pallas as pl
from jax.experimental.pallas import tpu as pltpu
from jax.experimental.pallas import tpu_sc as plsc

H = 256

NHI = 81
BAND = 256
NW = 32
CHUNK = 64
DEPTH = 4
AHEAD = 2


def _tbl_body(w0, w1, w2, w3, w4, w5, w6, w7, w8, out):
    i = pl.program_id(0)
    acc = w0[...]
    for wref in (w1, w2, w3, w4):
        w = wref[...]
        acc = jnp.concatenate(
            [acc + w[0:1, :], acc + w[1:2, :], acc + w[2:3, :]], axis=0)
    d7 = i % 3
    d8 = i // 3
    w7v = w7[...]
    w8v = w8[...]
    hi = (jnp.where(d7 == 0, w7v[0:1, :],
                    jnp.where(d7 == 1, w7v[1:2, :], w7v[2:3, :])) +
          jnp.where(d8 == 0, w8v[0:1, :],
                    jnp.where(d8 == 1, w8v[1:2, :], w8v[2:3, :])))
    acc = acc + hi
    w5v = w5[...]
    w6v = w6[...]
    pad = jnp.zeros((BAND - 243, H), jnp.float32)
    bands = []
    for m in range(9):
        row = w5v[m % 3:m % 3 + 1, :] + w6v[m // 3:m // 3 + 1, :]
        bands += [acc + row, pad]
    out[...] = jnp.concatenate(bands, axis=0) * 16.0


def _build_table(ws, interpret=False):
    return pl.pallas_call(
        _tbl_body,
        grid=(NHI // 9,),
        in_specs=[pl.BlockSpec((3, H), lambda i: (0, 0))] * 9,
        out_specs=pl.BlockSpec((9 * BAND, H), lambda i: (i, 0)),
        out_shape=jax.ShapeDtypeStruct((NHI * BAND, H), jnp.float32),
        interpret=interpret,
    )(*ws)


def _sc_body(n_tok, i0, i1, i2, i3, i4, i5, i6, i7, i8, table, out,
             b0, b1, b2, b3, b4, b5, b6, b7, b8, codes,
             rows0, rows1, rows2, rows3,
             isem, gsem0, gsem1, gsem2, gsem3, ssem0, ssem1, ssem2, ssem3):
    per_w = n_tok // NW
    nchunk = per_w // CHUNK
    wid = lax.axis_index("s") * 2 + lax.axis_index("c")
    base = wid * per_w
    t_len = i0.shape[1]
    rows_per_w = per_w // t_len
    row0 = wid * rows_per_w
    bufs = (b0, b1, b2, b3, b4, b5, b6, b7, b8)
    stages = []
    for src, dst in zip((i0, i1, i2, i3, i4, i5, i6, i7, i8), bufs):
        for r in range(rows_per_w):
            stages.append(pltpu.async_copy(
                src.at[row0 + r], dst.at[pl.ds(r * t_len, t_len)], isem))
    for h in stages:
        h.wait()

    rows = (rows0, rows1, rows2, rows3)
    gsem = (gsem0, gsem1, gsem2, gsem3)
    ssem = (ssem0, ssem1, ssem2, ssem3)

    def code_chunk(g):
        for j in range(CHUNK // 16):
            o = g * CHUNK + j * 16
            d = [b[pl.ds(o, 16)] for b in bufs]
            hi = ((d[8] * 3 + d[7]) * 3 + d[6]) * 3 + d[5]
            lo = (((d[4] * 3 + d[3]) * 3 + d[2]) * 3 + d[1]) * 3 + d[0]
            codes[pl.ds(o, 16)] = hi * BAND + lo

    def gather_start(g, s):
        pltpu.async_copy(table.at[codes.at[pl.ds(g * CHUNK, CHUNK)]],
                         rows[s], gsem[s])

    def gather_wait(g, s):
        pltpu.make_async_copy(table.at[codes.at[pl.ds(g * CHUNK, CHUNK)]],
                              rows[s], gsem[s]).wait()

    def scatter_start(g, s):
        pltpu.async_copy(rows[s], out.at[pl.ds(base + g * CHUNK, CHUNK)],
                         ssem[s])

    def scatter_wait(g, s):
        pltpu.make_async_copy(rows[s], out.at[pl.ds(base + g * CHUNK, CHUNK)],
                              ssem[s]).wait()

    for g in range(AHEAD):
        code_chunk(g)
        gather_start(g, g % DEPTH)

    @pl.loop(0, nchunk, step=DEPTH)
    def chunk_loop(k):
        for b in range(DEPTH):
            g = k + b
            s = b
            gather_wait(g, s)
            scatter_start(g, s)
            nxt = g + AHEAD
            s2 = (b + AHEAD) % DEPTH

            @pl.when(nxt < nchunk)
            def _():
                code_chunk(nxt)

                @pl.when(nxt - DEPTH >= 0)
                def _():
                    scatter_wait(nxt - DEPTH, s2)

                gather_start(nxt, s2)

    for s in range(DEPTH):
        scatter_wait(nchunk - DEPTH + s, s)


def _sc_lookup(idxs, table, interpret=False):
    n_tok = idxs[0].shape[0] * idxs[0].shape[1]
    per_w = n_tok // NW
    mesh = plsc.VectorSubcoreMesh(core_axis_name="c", subcore_axis_name="s")
    scratch = [pltpu.VMEM((per_w,), jnp.int32) for _ in range(9)]
    scratch += [pltpu.VMEM((per_w,), jnp.int32)]
    scratch += [pltpu.VMEM((CHUNK, H), jnp.float32) for _ in range(DEPTH)]
    scratch += [pltpu.SemaphoreType.DMA for _ in range(2 * DEPTH + 1)]
    fn = pl.kernel(
        functools.partial(_sc_body, n_tok),
        out_type=jax.ShapeDtypeStruct((n_tok, H), jnp.float32),
        mesh=mesh,
        scratch_types=scratch,
        interpret=interpret,
    )
    return fn(*idxs, table)


def kernel(mix, falsetto, breathy, bubble, strong, weak, pharyngeal,
           vibrato, glissando,
           W_mix, W_falsetto, W_breathy, W_bubble, W_strong, W_weak,
           W_pharyngeal, W_vibrato, W_glissando):
    b, t = mix.shape
    idxs = [mix, falsetto, breathy, bubble, strong, weak, pharyngeal,
            vibrato, glissando]
    ws = (W_mix, W_falsetto, W_breathy, W_bubble, W_strong, W_weak,
          W_pharyngeal, W_vibrato, W_glissando)
    table = _build_table(ws)
    out = _sc_lookup(idxs, table)
    return out.reshape(b, t, H)


# speedup vs baseline: 1.5833x; 1.0081x over previous
import functools
import jax
import jax.numpy as jnp
import numpy as np
from jax import lax
from jax.experimental import pallas as pl
from jax.experimental.pallas import tpu as pltpu
from jax.experimental.pallas import tpu_sc as plsc

H = 256

NHI = 81
BAND = 256
NW = 32
CHUNK = 32
DEPTH = 8
AHEAD = 4


def _tbl_body(w0, w1, w2, w3, w4, w5, w6, w7, w8, out):
    i = pl.program_id(0)
    acc = w0[...]
    for wref in (w1, w2, w3, w4):
        w = wref[...]
        acc = jnp.concatenate(
            [acc + w[0:1, :], acc + w[1:2, :], acc + w[2:3, :]], axis=0)
    d7 = i % 3
    d8 = i // 3
    w7v = w7[...]
    w8v = w8[...]
    hi = (jnp.where(d7 == 0, w7v[0:1, :],
                    jnp.where(d7 == 1, w7v[1:2, :], w7v[2:3, :])) +
          jnp.where(d8 == 0, w8v[0:1, :],
                    jnp.where(d8 == 1, w8v[1:2, :], w8v[2:3, :])))
    acc = acc + hi
    w5v = w5[...]
    w6v = w6[...]
    pad = jnp.zeros((BAND - 243, H), jnp.float32)
    bands = []
    for m in range(9):
        row = w5v[m % 3:m % 3 + 1, :] + w6v[m // 3:m // 3 + 1, :]
        bands += [acc + row, pad]
    out[...] = jnp.concatenate(bands, axis=0) * 16.0


def _build_table(ws, interpret=False):
    return pl.pallas_call(
        _tbl_body,
        grid=(NHI // 9,),
        in_specs=[pl.BlockSpec((3, H), lambda i: (0, 0))] * 9,
        out_specs=pl.BlockSpec((9 * BAND, H), lambda i: (i, 0)),
        out_shape=jax.ShapeDtypeStruct((NHI * BAND, H), jnp.float32),
        interpret=interpret,
    )(*ws)


def _sc_body(n_tok, i0, i1, i2, i3, i4, i5, i6, i7, i8, table, out,
             b0, b1, b2, b3, b4, b5, b6, b7, b8, codes,
             rows0, rows1, rows2, rows3, rows4, rows5, rows6, rows7,
             isem, gsem0, gsem1, gsem2, gsem3, gsem4, gsem5, gsem6, gsem7,
             ssem0, ssem1, ssem2, ssem3, ssem4, ssem5, ssem6, ssem7):
    per_w = n_tok // NW
    nchunk = per_w // CHUNK
    wid = lax.axis_index("s") * 2 + lax.axis_index("c")
    base = wid * per_w
    t_len = i0.shape[1]
    rows_per_w = per_w // t_len
    row0 = wid * rows_per_w
    bufs = (b0, b1, b2, b3, b4, b5, b6, b7, b8)
    stages = []
    for src, dst in zip((i0, i1, i2, i3, i4, i5, i6, i7, i8), bufs):
        for r in range(rows_per_w):
            stages.append(pltpu.async_copy(
                src.at[row0 + r], dst.at[pl.ds(r * t_len, t_len)], isem))
    for h in stages:
        h.wait()

    rows = (rows0, rows1, rows2, rows3, rows4, rows5, rows6, rows7)
    gsem = (gsem0, gsem1, gsem2, gsem3, gsem4, gsem5, gsem6, gsem7)
    ssem = (ssem0, ssem1, ssem2, ssem3, ssem4, ssem5, ssem6, ssem7)

    def code_chunk(g):
        for j in range(CHUNK // 16):
            o = g * CHUNK + j * 16
            d = [b[pl.ds(o, 16)] for b in bufs]
            hi = ((d[8] * 3 + d[7]) * 3 + d[6]) * 3 + d[5]
            lo = (((d[4] * 3 + d[3]) * 3 + d[2]) * 3 + d[1]) * 3 + d[0]
            codes[pl.ds(o, 16)] = hi * BAND + lo

    def gather_start(g, s):
        pltpu.async_copy(table.at[codes.at[pl.ds(g * CHUNK, CHUNK)]],
                         rows[s], gsem[s])

    def gather_wait(g, s):
        pltpu.make_async_copy(table.at[codes.at[pl.ds(g * CHUNK, CHUNK)]],
                              rows[s], gsem[s]).wait()

    def scatter_start(g, s):
        pltpu.async_copy(rows[s], out.at[pl.ds(base + g * CHUNK, CHUNK)],
                         ssem[s])

    def scatter_wait(g, s):
        pltpu.make_async_copy(rows[s], out.at[pl.ds(base + g * CHUNK, CHUNK)],
                              ssem[s]).wait()

    for g in range(AHEAD):
        code_chunk(g)
        gather_start(g, g % DEPTH)

    @pl.loop(0, nchunk, step=DEPTH)
    def chunk_loop(k):
        for b in range(DEPTH):
            g = k + b
            s = b
            gather_wait(g, s)
            scatter_start(g, s)
            nxt = g + AHEAD
            s2 = (b + AHEAD) % DEPTH

            @pl.when(nxt < nchunk)
            def _():
                code_chunk(nxt)

                @pl.when(nxt - DEPTH >= 0)
                def _():
                    scatter_wait(nxt - DEPTH, s2)

                gather_start(nxt, s2)

    for s in range(DEPTH):
        scatter_wait(nchunk - DEPTH + s, s)


def _sc_lookup(idxs, table, interpret=False):
    n_tok = idxs[0].shape[0] * idxs[0].shape[1]
    per_w = n_tok // NW
    mesh = plsc.VectorSubcoreMesh(core_axis_name="c", subcore_axis_name="s")
    scratch = [pltpu.VMEM((per_w,), jnp.int32) for _ in range(9)]
    scratch += [pltpu.VMEM((per_w,), jnp.int32)]
    scratch += [pltpu.VMEM((CHUNK, H), jnp.float32) for _ in range(DEPTH)]
    scratch += [pltpu.SemaphoreType.DMA for _ in range(2 * DEPTH + 1)]
    fn = pl.kernel(
        functools.partial(_sc_body, n_tok),
        out_type=jax.ShapeDtypeStruct((n_tok, H), jnp.float32),
        mesh=mesh,
        scratch_types=scratch,
        interpret=interpret,
    )
    return fn(*idxs, table)


def kernel(mix, falsetto, breathy, bubble, strong, weak, pharyngeal,
           vibrato, glissando,
           W_mix, W_falsetto, W_breathy, W_bubble, W_strong, W_weak,
           W_pharyngeal, W_vibrato, W_glissando):
    b, t = mix.shape
    idxs = [mix, falsetto, breathy, bubble, strong, weak, pharyngeal,
            vibrato, glissando]
    ws = (W_mix, W_falsetto, W_breathy, W_bubble, W_strong, W_weak,
          W_pharyngeal, W_vibrato, W_glissando)
    table = _build_table(ws)
    out = _sc_lookup(idxs, table)
    return out.reshape(b, t, H)
